# Initial kernel scaffold; baseline (speedup 1.0000x reference)
import jax
import jax.numpy as jnp
from jax.experimental import pallas as pl


def kernel(v, s, r, W1, b1, W2, b2, Wr, br):
    raise NotImplementedError("write your pallas kernel here")


# speedup vs baseline: 10.4257x; 10.4257x over previous
import functools

import jax
import jax.numpy as jnp
from jax import lax
from jax.experimental import pallas as pl
from jax.experimental.pallas import tpu as pltpu
from jax.experimental.pallas import tpu_sc as plsc

NF = 128
NRBF = 20
CUT = 5.0
_ET = 2000
_NT = 2000
_C = 128
_NSUB = 16
_EPS = 1e-8


def _edge_body(rx, ry, rz, wr, br, out):
    x_ = rx[...]
    y_ = ry[...]
    z_ = rz[...]
    nz = jnp.abs(z_)
    n = lax.broadcasted_iota(jnp.int32, (1, NRBF), 1).astype(jnp.float32) + 1.0
    rbf = jnp.sin(n * (jnp.pi / CUT) * nz) / (nz + _EPS)
    xl = jnp.dot(rbf, wr[...], preferred_element_type=jnp.float32) + br[...]
    w = 0.5 * (jnp.cos(jnp.pi * xl / CUT) + 1.0)
    w = w * jnp.where(xl < CUT, 1.0, 0.0)
    inv = 1.0 / (jnp.sqrt(x_ * x_ + y_ * y_ + z_ * z_) + _EPS)
    p3 = w[:, 2 * NF:]
    out[0] = w[:, :NF]
    out[1] = w[:, NF:2 * NF]
    out[2] = p3 * (x_ * inv)
    out[3] = p3 * (y_ * inv)
    out[4] = p3 * (z_ * inv)


def _edge_payload(rx, ry, rz, Wr, br2):
    E = rx.shape[0]
    return pl.pallas_call(
        _edge_body,
        grid=(E // _ET,),
        in_specs=[
            pl.BlockSpec((_ET, 1), lambda i: (i, 0)),
            pl.BlockSpec((_ET, 1), lambda i: (i, 0)),
            pl.BlockSpec((_ET, 1), lambda i: (i, 0)),
            pl.BlockSpec((NRBF, 3 * NF), lambda i: (0, 0)),
            pl.BlockSpec((1, 3 * NF), lambda i: (0, 0)),
        ],
        out_specs=pl.BlockSpec((5, _ET, NF), lambda i: (0, i, 0)),
        out_shape=jax.ShapeDtypeStruct((5, E, NF), jnp.float32),
    )(rx, ry, rz, Wr, br2)


def _make_sc(N, E):
    chunks = E // _C
    iters = -(-chunks // _NSUB)
    NP = -(-N // (_NSUB * 128)) * (_NSUB * 128)
    rps = NP // _NSUB
    zr = 128
    ncp = rps // zr
    mesh = plsc.VectorSubcoreMesh(core_axis_name="c", subcore_axis_name="s")

    @functools.partial(
        pl.kernel,
        out_type=jax.ShapeDtypeStruct((5 * NP, NF), jnp.float32),
        mesh=mesh,
        scratch_types=[
            pltpu.VMEM((_C,), jnp.int32),
            pltpu.VMEM((_C, NF), jnp.float32),
            pltpu.VMEM((zr, NF), jnp.float32),
            pltpu.VMEM_SHARED((NP, NF), jnp.float32),
        ],
    )
    def sc_k(pay_hbm, js_hbm, out_hbm, idx_v, pay_v, z_v, acc):
        c = lax.axis_index("c")
        sid = lax.axis_index("s")

        def zrow(i, carry):
            for jj in range(NF // 16):
                z_v[i, pl.ds(jj * 16, 16)] = jnp.zeros((16,), jnp.float32)
            return carry

        lax.fori_loop(0, zr, zrow, 0)

        for gi in range(3):
            g = gi + 3 * c
            active = jnp.logical_or(c == 0, gi < 2)

            @pl.when(active)
            def _():
                for t in range(ncp):
                    pltpu.sync_copy(z_v, acc.at[pl.ds(sid * rps + t * zr, zr)])
                plsc.subcore_barrier()

                def chunk(i, carry):
                    k = sid + i * _NSUB

                    @pl.when(k < chunks)
                    def _():
                        base = k * _C
                        pltpu.sync_copy(js_hbm.at[pl.ds(base, _C)], idx_v)
                        pltpu.sync_copy(pay_hbm.at[pl.ds(g * E + base, _C)], pay_v)
                        pltpu.sync_copy(pay_v, acc.at[idx_v], add=True)

                    return carry

                lax.fori_loop(0, iters, chunk, 0)
                plsc.subcore_barrier()
                for t in range(ncp):
                    o = sid * rps + t * zr
                    pltpu.sync_copy(acc.at[pl.ds(o, zr)],
                                    out_hbm.at[pl.ds(g * NP + o, zr)])
                plsc.subcore_barrier()

    return sc_k


def _combine_body(s_r, v3_r, acc_r, w1, b1, w2, b2, dv, ds):
    t = jnp.dot(s_r[...], w1[...], preferred_element_type=jnp.float32) + b1[...]
    h = t * (1.0 / (1.0 + jnp.exp(-t)))
    phi = jnp.dot(h, w2[...], preferred_element_type=jnp.float32) + b2[...]
    p1 = phi[:, :NF]
    p2 = phi[:, NF:2 * NF]
    p3 = phi[:, 2 * NF:]
    ds[...] = p2 * acc_r[1]
    a = p1 * acc_r[0]
    for d in range(3):
        dv[d] = v3_r[d] * a + p3 * acc_r[2 + d]


def _combine(s, v3, acc5, W1, b12, W2, b22):
    N = s.shape[0]
    return pl.pallas_call(
        _combine_body,
        grid=(N // _NT,),
        in_specs=[
            pl.BlockSpec((_NT, NF), lambda i: (i, 0)),
            pl.BlockSpec((3, _NT, NF), lambda i: (0, i, 0)),
            pl.BlockSpec((5, _NT, NF), lambda i: (0, i, 0)),
            pl.BlockSpec((NF, NF), lambda i: (0, 0)),
            pl.BlockSpec((1, NF), lambda i: (0, 0)),
            pl.BlockSpec((NF, 3 * NF), lambda i: (0, 0)),
            pl.BlockSpec((1, 3 * NF), lambda i: (0, 0)),
        ],
        out_specs=[
            pl.BlockSpec((3, _NT, NF), lambda i: (0, i, 0)),
            pl.BlockSpec((_NT, NF), lambda i: (i, 0)),
        ],
        out_shape=[
            jax.ShapeDtypeStruct((3, N, NF), jnp.float32),
            jax.ShapeDtypeStruct((N, NF), jnp.float32),
        ],
    )(s, v3, acc5, W1, b12, W2, b22)


def kernel(v, s, r, W1, b1, W2, b2, Wr, br):
    N = s.shape[0]
    E = r.shape[0]
    js = r[:, 1].astype(jnp.int32)
    rx = r[:, 2:3]
    ry = r[:, 3:4]
    rz = r[:, 4:5]
    NP = -(-N // (_NSUB * 128)) * (_NSUB * 128)
    pay = _edge_payload(rx, ry, rz, Wr, br.reshape(1, -1))
    acc = _make_sc(N, E)(pay.reshape(5 * E, NF), js)
    v3 = jnp.transpose(v, (2, 0, 1))
    dv3, ds = _combine(s, v3, acc.reshape(5, NP, NF), W1,
                       b1.reshape(1, -1), W2, b2.reshape(1, -1))
    return jnp.transpose(dv3, (1, 2, 0)), ds
